# baseline (speedup 1.0000x reference)
import jax
import jax.numpy as jnp
from jax import lax
from jax.experimental import pallas as pl
from jax.experimental.pallas import tpu as pltpu
from jax.experimental.pallas import tpu_sc as plsc

E = 320000
D = 128
G = 64
L = 16
DC = D // L

E_TC = 230400
E_SC = E - E_TC

NC = 2
NS = 16
NW = NC * NS
ROWS_PER_W = E_SC // NW
BLK = 80
NBLK = ROWS_PER_W // BLK
RING = 5
assert ROWS_PER_W * NW == E_SC and NBLK * BLK == ROWS_PER_W
assert NBLK % RING == 0 and BLK % L == 0

TBLK = 3200
TNBLK = E_TC // TBLK
assert TNBLK * TBLK == E_TC


def _gate_from_partials(p, bv):
    rp = lax.rev(p, (0,))
    tot = plsc.cumsum(p) + lax.rev(plsc.cumsum(rp), (0,)) - p
    z2 = jnp.minimum((tot + bv) * 2.0, 30.0)
    t = jnp.exp(z2)
    return (t - 1.0) / (t + 1.0)


def _dot_partials(xk, wk):
    a = xk[0] * wk[0]
    b = xk[1] * wk[1]
    for k in range(2, DC, 2):
        a = a + xk[k] * wk[k]
        b = b + xk[k + 1] * wk[k + 1]
    return a + b


def _sc_body(x_hbm, ids_hbm, wb_hbm, out_hbm,
             xb, idsbuf, wbuf, lacc, iotabuf, acc_sh, sem_in, sem_ids):
    c = lax.axis_index("c")
    s = lax.axis_index("s")
    wid = s * NC + c
    base = E_TC + wid * ROWS_PER_W

    def in_x(b, j):
        return pltpu.make_async_copy(
            x_hbm.at[pl.ds(base + b * BLK, BLK), :], xb.at[j], sem_in.at[j])

    pltpu.make_async_copy(ids_hbm.at[pl.ds(base, ROWS_PER_W)], idsbuf,
                          sem_ids).start()
    for j in range(RING - 1):
        in_x(j, j).start()

    pltpu.sync_copy(wb_hbm, wbuf)
    wk = [wbuf[pl.ds(k * L, L)] for k in range(DC)]
    bv = wbuf[pl.ds(D, L)]

    zero = jnp.zeros((L,), jnp.float32)

    def zero_one(i, _):
        for k in range(DC):
            lacc[i, pl.ds(k * L, L)] = zero
        return 0
    lax.fori_loop(0, G, zero_one, 0)
    for q in range(G // L):
        iotabuf[pl.ds(q * L, L)] = lax.iota(jnp.int32, L) + (q * L)

    @pl.when(s == 0)
    def _init():
        pltpu.sync_copy(lacc, acc_sh)

    plsc.subcore_barrier()
    pltpu.make_async_copy(ids_hbm.at[pl.ds(base, ROWS_PER_W)], idsbuf,
                          sem_ids).wait()

    @pl.loop(0, NBLK, step=RING)
    def _blocks(b0):
        for j in range(RING):
            b = b0 + j
            j4 = (j + RING - 1) % RING

            @pl.when(b + (RING - 1) < NBLK)
            def _prefetch():
                in_x(b + (RING - 1), j4).start()

            in_x(b, j).wait()
            xs = xb.at[j]
            boff = b * BLK

            gfv = idsbuf[pl.ds(boff, L)]
            glv = idsbuf[pl.ds(boff + BLK - L, L)]
            gf = gfv[0]
            gl = glv[L - 1]

            @pl.when(gf == gl)
            def _fast():
                def row(r, acc):
                    xk = [xs[r, pl.ds(k * L, L)] for k in range(DC)]
                    gate = _gate_from_partials(_dot_partials(xk, wk), bv)
                    return tuple(acc[k] + xk[k] * gate for k in range(DC))

                acc = lax.fori_loop(0, BLK, row, (zero,) * DC, unroll=16)
                for k in range(DC):
                    plsc.addupdate(lacc.at[gf, pl.ds(k * L, L)], acc[k])

            @pl.when(gf != gl)
            def _slow():
                def grp(i, _):
                    gv = idsbuf[pl.ds(boff + i * L, L)]
                    for u in range(L):
                        r = i * L + u
                        g = gv[u]
                        xk = [xs[r, pl.ds(k * L, L)] for k in range(DC)]
                        gate = _gate_from_partials(_dot_partials(xk, wk), bv)
                        for k in range(DC):
                            plsc.addupdate(lacc.at[g, pl.ds(k * L, L)],
                                           xk[k] * gate)
                    return 0
                lax.fori_loop(0, BLK // L, grp, 0)

    pltpu.sync_copy(lacc, acc_sh.at[iotabuf], add=True)
    plsc.subcore_barrier()

    @pl.when(s == 0)
    def _writeout():
        pltpu.sync_copy(acc_sh, out_hbm.at[c])


def _tc_body(x_ref, ids_ref, wrep_ref, b_ref, out_ref):
    i = pl.program_id(0)

    @pl.when(i == 0)
    def _init():
        out_ref[...] = jnp.zeros((G, D), jnp.float32)

    x = x_ref[...]
    z = jnp.dot(x, wrep_ref[...], preferred_element_type=jnp.float32)
    gate = jnp.tanh(z + b_ref[0])
    weighted = x * gate
    ids = ids_ref[0]
    rows = lax.broadcasted_iota(jnp.int32, (G, TBLK), 0)
    onehot = (rows == ids).astype(jnp.float32)
    out_ref[...] += jnp.dot(onehot, weighted,
                            preferred_element_type=jnp.float32)


@jax.jit
def _cys_readout(edge_feats, ids_i32, wb, ids_tc3, wrep, bvec):
    mesh = plsc.VectorSubcoreMesh(core_axis_name="c", subcore_axis_name="s")
    sc_partials = pl.kernel(
        _sc_body,
        out_type=jax.ShapeDtypeStruct((NC, G, D), jnp.float32),
        mesh=mesh,
        compiler_params=pltpu.CompilerParams(needs_layout_passes=False),
        scratch_types=[
            pltpu.VMEM((RING, BLK, D), jnp.float32),
            pltpu.VMEM((ROWS_PER_W,), jnp.int32),
            pltpu.VMEM((D + L,), jnp.float32),
            pltpu.VMEM((G, D), jnp.float32),
            pltpu.VMEM((G,), jnp.int32),
            pltpu.VMEM_SHARED((G, D), jnp.float32),
            pltpu.SemaphoreType.DMA((RING,)),
            pltpu.SemaphoreType.DMA,
        ],
        cost_estimate=pl.CostEstimate(
            flops=4 * E_SC * D,
            bytes_accessed=E_SC * D * 4,
            transcendentals=E_SC),
    )(edge_feats, ids_i32, wb)

    tc_out = pl.pallas_call(
        _tc_body,
        grid=(TNBLK,),
        in_specs=[
            pl.BlockSpec((TBLK, D), lambda i: (i, 0)),
            pl.BlockSpec((1, 1, TBLK), lambda i: (i, 0, 0)),
            pl.BlockSpec((D, D), lambda i: (0, 0)),
            pl.BlockSpec(memory_space=pltpu.SMEM),
        ],
        out_specs=pl.BlockSpec((G, D), lambda i: (0, 0)),
        out_shape=jax.ShapeDtypeStruct((G, D), jnp.float32),
        compiler_params=pltpu.CompilerParams(
            dimension_semantics=("arbitrary",)),
        cost_estimate=pl.CostEstimate(
            flops=2 * E_TC * D * (D + G),
            bytes_accessed=E_TC * D * 4,
            transcendentals=E_TC * D),
    )(edge_feats, ids_tc3, wrep, bvec)

    return sc_partials[0] + sc_partials[1] + tc_out


def kernel(edge_feats, edge_graph_ids, W, b):
    ids_i32 = edge_graph_ids.astype(jnp.int32)
    wb = jnp.concatenate([W[:, 0], jnp.broadcast_to(b, (L,))]).astype(jnp.float32)
    ids_tc3 = ids_i32[:E_TC].reshape(TNBLK, 1, TBLK)
    wrep = jnp.broadcast_to(W, (D, D)).astype(jnp.float32)
    bvec = b.astype(jnp.float32)
    return _cys_readout(edge_feats, ids_i32, wb, ids_tc3, wrep, bvec)


# speedup vs baseline: 1.0305x; 1.0305x over previous
import jax
import jax.numpy as jnp
from jax import lax
from jax.experimental import pallas as pl
from jax.experimental.pallas import tpu as pltpu
from jax.experimental.pallas import tpu_sc as plsc

E = 320000
D = 128
G = 64
L = 16
DC = D // L

E_TC = 217600
E_SC = E - E_TC

NC = 2
NS = 16
NW = NC * NS
ROWS_PER_W = E_SC // NW
BLK = 80
NBLK = ROWS_PER_W // BLK
RING = 5
assert ROWS_PER_W * NW == E_SC and NBLK * BLK == ROWS_PER_W
assert NBLK % RING == 0 and BLK % L == 0

TBLK = 3200
TNBLK = E_TC // TBLK
assert TNBLK * TBLK == E_TC


def _gate_from_partials(p, bv):
    rp = lax.rev(p, (0,))
    tot = plsc.cumsum(p) + lax.rev(plsc.cumsum(rp), (0,)) - p
    z2 = jnp.minimum((tot + bv) * 2.0, 30.0)
    t = jnp.exp(z2)
    return (t - 1.0) / (t + 1.0)


def _dot_partials(xk, wk):
    a = xk[0] * wk[0]
    b = xk[1] * wk[1]
    for k in range(2, DC, 2):
        a = a + xk[k] * wk[k]
        b = b + xk[k + 1] * wk[k + 1]
    return a + b


def _sc_body(x_hbm, ids_hbm, wb_hbm, out_hbm,
             xb, idsbuf, wbuf, lacc, iotabuf, acc_sh, sem_in, sem_ids):
    c = lax.axis_index("c")
    s = lax.axis_index("s")
    wid = s * NC + c
    base = E_TC + wid * ROWS_PER_W

    def in_x(b, j):
        return pltpu.make_async_copy(
            x_hbm.at[pl.ds(base + b * BLK, BLK), :], xb.at[j], sem_in.at[j])

    pltpu.make_async_copy(ids_hbm.at[pl.ds(base, ROWS_PER_W)], idsbuf,
                          sem_ids).start()
    for j in range(RING - 1):
        in_x(j, j).start()

    pltpu.sync_copy(wb_hbm, wbuf)
    wk = [wbuf[pl.ds(k * L, L)] for k in range(DC)]
    bv = wbuf[pl.ds(D, L)]

    zero = jnp.zeros((L,), jnp.float32)

    def zero_one(i, _):
        for k in range(DC):
            lacc[i, pl.ds(k * L, L)] = zero
        return 0
    lax.fori_loop(0, G, zero_one, 0)
    for q in range(G // L):
        iotabuf[pl.ds(q * L, L)] = lax.iota(jnp.int32, L) + (q * L)

    @pl.when(s == 0)
    def _init():
        pltpu.sync_copy(lacc, acc_sh)

    plsc.subcore_barrier()
    pltpu.make_async_copy(ids_hbm.at[pl.ds(base, ROWS_PER_W)], idsbuf,
                          sem_ids).wait()

    @pl.loop(0, NBLK, step=RING)
    def _blocks(b0):
        for j in range(RING):
            b = b0 + j
            j4 = (j + RING - 1) % RING

            @pl.when(b + (RING - 1) < NBLK)
            def _prefetch():
                in_x(b + (RING - 1), j4).start()

            in_x(b, j).wait()
            xs = xb.at[j]
            boff = b * BLK

            gfv = idsbuf[pl.ds(boff, L)]
            glv = idsbuf[pl.ds(boff + BLK - L, L)]
            gf = gfv[0]
            gl = glv[L - 1]

            @pl.when(gf == gl)
            def _fast():
                def row(r, acc):
                    xk = [xs[r, pl.ds(k * L, L)] for k in range(DC)]
                    gate = _gate_from_partials(_dot_partials(xk, wk), bv)
                    return tuple(acc[k] + xk[k] * gate for k in range(DC))

                acc = lax.fori_loop(0, BLK, row, (zero,) * DC, unroll=16)
                for k in range(DC):
                    plsc.addupdate(lacc.at[gf, pl.ds(k * L, L)], acc[k])

            @pl.when(gf != gl)
            def _slow():
                def grp(i, _):
                    gv = idsbuf[pl.ds(boff + i * L, L)]
                    for u in range(L):
                        r = i * L + u
                        g = gv[u]
                        xk = [xs[r, pl.ds(k * L, L)] for k in range(DC)]
                        gate = _gate_from_partials(_dot_partials(xk, wk), bv)
                        for k in range(DC):
                            plsc.addupdate(lacc.at[g, pl.ds(k * L, L)],
                                           xk[k] * gate)
                    return 0
                lax.fori_loop(0, BLK // L, grp, 0)

    pltpu.sync_copy(lacc, acc_sh.at[iotabuf], add=True)
    plsc.subcore_barrier()

    @pl.when(s == 0)
    def _writeout():
        pltpu.sync_copy(acc_sh, out_hbm.at[c])


def _tc_body(x_ref, ids_ref, wrep_ref, b_ref, out_ref):
    i = pl.program_id(0)

    @pl.when(i == 0)
    def _init():
        out_ref[...] = jnp.zeros((G, D), jnp.float32)

    x = x_ref[...]
    z = jnp.dot(x, wrep_ref[...], preferred_element_type=jnp.float32)
    gate = jnp.tanh(z + b_ref[0])
    weighted = x * gate
    ids = ids_ref[0]
    rows = lax.broadcasted_iota(jnp.int32, (G, TBLK), 0)
    onehot = (rows == ids).astype(jnp.float32)
    out_ref[...] += jnp.dot(onehot, weighted,
                            preferred_element_type=jnp.float32)


@jax.jit
def _cys_readout(edge_feats, ids_i32, wb, ids_tc3, wrep, bvec):
    mesh = plsc.VectorSubcoreMesh(core_axis_name="c", subcore_axis_name="s")
    sc_partials = pl.kernel(
        _sc_body,
        out_type=jax.ShapeDtypeStruct((NC, G, D), jnp.float32),
        mesh=mesh,
        compiler_params=pltpu.CompilerParams(needs_layout_passes=False),
        scratch_types=[
            pltpu.VMEM((RING, BLK, D), jnp.float32),
            pltpu.VMEM((ROWS_PER_W,), jnp.int32),
            pltpu.VMEM((D + L,), jnp.float32),
            pltpu.VMEM((G, D), jnp.float32),
            pltpu.VMEM((G,), jnp.int32),
            pltpu.VMEM_SHARED((G, D), jnp.float32),
            pltpu.SemaphoreType.DMA((RING,)),
            pltpu.SemaphoreType.DMA,
        ],
        cost_estimate=pl.CostEstimate(
            flops=4 * E_SC * D,
            bytes_accessed=E_SC * D * 4,
            transcendentals=E_SC),
    )(edge_feats, ids_i32, wb)

    tc_out = pl.pallas_call(
        _tc_body,
        grid=(TNBLK,),
        in_specs=[
            pl.BlockSpec((TBLK, D), lambda i: (i, 0)),
            pl.BlockSpec((1, 1, TBLK), lambda i: (i, 0, 0)),
            pl.BlockSpec((D, D), lambda i: (0, 0)),
            pl.BlockSpec(memory_space=pltpu.SMEM),
        ],
        out_specs=pl.BlockSpec((G, D), lambda i: (0, 0)),
        out_shape=jax.ShapeDtypeStruct((G, D), jnp.float32),
        compiler_params=pltpu.CompilerParams(
            dimension_semantics=("arbitrary",)),
        cost_estimate=pl.CostEstimate(
            flops=2 * E_TC * D * (D + G),
            bytes_accessed=E_TC * D * 4,
            transcendentals=E_TC * D),
    )(edge_feats, ids_tc3, wrep, bvec)

    return sc_partials[0] + sc_partials[1] + tc_out


def kernel(edge_feats, edge_graph_ids, W, b):
    ids_i32 = edge_graph_ids.astype(jnp.int32)
    wb = jnp.concatenate([W[:, 0], jnp.broadcast_to(b, (L,))]).astype(jnp.float32)
    ids_tc3 = ids_i32[:E_TC].reshape(TNBLK, 1, TBLK)
    wrep = jnp.broadcast_to(W, (D, D)).astype(jnp.float32)
    bvec = b.astype(jnp.float32)
    return _cys_readout(edge_feats, ids_i32, wb, ids_tc3, wrep, bvec)


# speedup vs baseline: 1.0371x; 1.0064x over previous
import jax
import jax.numpy as jnp
from jax import lax
from jax.experimental import pallas as pl
from jax.experimental.pallas import tpu as pltpu
from jax.experimental.pallas import tpu_sc as plsc

E = 320000
D = 128
G = 64
L = 16
DC = D // L

E_TC = 217600
E_SC = E - E_TC

NC = 2
NS = 16
NW = NC * NS
ROWS_PER_W = E_SC // NW
BLK = 80
NBLK = ROWS_PER_W // BLK
RING = 5
assert ROWS_PER_W * NW == E_SC and NBLK * BLK == ROWS_PER_W
assert NBLK % RING == 0 and BLK % L == 0

TBLK = 4352
TNBLK = E_TC // TBLK
assert TNBLK * TBLK == E_TC


def _gate_from_partials(p, bv):
    rp = lax.rev(p, (0,))
    tot = plsc.cumsum(p) + lax.rev(plsc.cumsum(rp), (0,)) - p
    z2 = jnp.minimum((tot + bv) * 2.0, 30.0)
    t = jnp.exp(z2)
    return (t - 1.0) / (t + 1.0)


def _dot_partials(xk, wk):
    a = xk[0] * wk[0]
    b = xk[1] * wk[1]
    for k in range(2, DC, 2):
        a = a + xk[k] * wk[k]
        b = b + xk[k + 1] * wk[k + 1]
    return a + b


def _sc_body(x_hbm, ids_hbm, wb_hbm, out_hbm,
             xb, idsbuf, wbuf, lacc, iotabuf, acc_sh, sem_in, sem_ids):
    c = lax.axis_index("c")
    s = lax.axis_index("s")
    wid = s * NC + c
    base = E_TC + wid * ROWS_PER_W

    def in_x(b, j):
        return pltpu.make_async_copy(
            x_hbm.at[pl.ds(base + b * BLK, BLK), :], xb.at[j], sem_in.at[j])

    pltpu.make_async_copy(ids_hbm.at[pl.ds(base, ROWS_PER_W)], idsbuf,
                          sem_ids).start()
    for j in range(RING - 1):
        in_x(j, j).start()

    pltpu.sync_copy(wb_hbm, wbuf)
    wk = [wbuf[pl.ds(k * L, L)] for k in range(DC)]
    bv = wbuf[pl.ds(D, L)]

    zero = jnp.zeros((L,), jnp.float32)

    def zero_one(i, _):
        for k in range(DC):
            lacc[i, pl.ds(k * L, L)] = zero
        return 0
    lax.fori_loop(0, G, zero_one, 0)
    for q in range(G // L):
        iotabuf[pl.ds(q * L, L)] = lax.iota(jnp.int32, L) + (q * L)

    @pl.when(s == 0)
    def _init():
        pltpu.sync_copy(lacc, acc_sh)

    plsc.subcore_barrier()
    pltpu.make_async_copy(ids_hbm.at[pl.ds(base, ROWS_PER_W)], idsbuf,
                          sem_ids).wait()

    @pl.loop(0, NBLK, step=RING)
    def _blocks(b0):
        for j in range(RING):
            b = b0 + j
            j4 = (j + RING - 1) % RING

            @pl.when(b + (RING - 1) < NBLK)
            def _prefetch():
                in_x(b + (RING - 1), j4).start()

            in_x(b, j).wait()
            xs = xb.at[j]
            boff = b * BLK

            gfv = idsbuf[pl.ds(boff, L)]
            glv = idsbuf[pl.ds(boff + BLK - L, L)]
            gf = gfv[0]
            gl = glv[L - 1]

            @pl.when(gf == gl)
            def _fast():
                def row(r, acc):
                    xk = [xs[r, pl.ds(k * L, L)] for k in range(DC)]
                    gate = _gate_from_partials(_dot_partials(xk, wk), bv)
                    return tuple(acc[k] + xk[k] * gate for k in range(DC))

                acc = lax.fori_loop(0, BLK, row, (zero,) * DC, unroll=16)
                for k in range(DC):
                    plsc.addupdate(lacc.at[gf, pl.ds(k * L, L)], acc[k])

            @pl.when(gf != gl)
            def _slow():
                def grp(i, _):
                    gv = idsbuf[pl.ds(boff + i * L, L)]
                    for u in range(L):
                        r = i * L + u
                        g = gv[u]
                        xk = [xs[r, pl.ds(k * L, L)] for k in range(DC)]
                        gate = _gate_from_partials(_dot_partials(xk, wk), bv)
                        for k in range(DC):
                            plsc.addupdate(lacc.at[g, pl.ds(k * L, L)],
                                           xk[k] * gate)
                    return 0
                lax.fori_loop(0, BLK // L, grp, 0)

    pltpu.sync_copy(lacc, acc_sh.at[iotabuf], add=True)
    plsc.subcore_barrier()

    @pl.when(s == 0)
    def _writeout():
        pltpu.sync_copy(acc_sh, out_hbm.at[c])


def _tc_body(x_ref, ids_ref, wrep_ref, b_ref, out_ref):
    i = pl.program_id(0)

    @pl.when(i == 0)
    def _init():
        out_ref[...] = jnp.zeros((G, D), jnp.float32)

    x = x_ref[...]
    z = jnp.dot(x, wrep_ref[...], preferred_element_type=jnp.float32)
    gate = jnp.tanh(z + b_ref[0])
    weighted = x * gate
    ids = ids_ref[0]
    rows = lax.broadcasted_iota(jnp.int32, (G, TBLK), 0)
    onehot = (rows == ids).astype(jnp.float32)
    out_ref[...] += jnp.dot(onehot, weighted,
                            preferred_element_type=jnp.float32)


@jax.jit
def _cys_readout(edge_feats, ids_i32, wb, ids_tc3, wrep, bvec):
    mesh = plsc.VectorSubcoreMesh(core_axis_name="c", subcore_axis_name="s")
    sc_partials = pl.kernel(
        _sc_body,
        out_type=jax.ShapeDtypeStruct((NC, G, D), jnp.float32),
        mesh=mesh,
        compiler_params=pltpu.CompilerParams(needs_layout_passes=False),
        scratch_types=[
            pltpu.VMEM((RING, BLK, D), jnp.float32),
            pltpu.VMEM((ROWS_PER_W,), jnp.int32),
            pltpu.VMEM((D + L,), jnp.float32),
            pltpu.VMEM((G, D), jnp.float32),
            pltpu.VMEM((G,), jnp.int32),
            pltpu.VMEM_SHARED((G, D), jnp.float32),
            pltpu.SemaphoreType.DMA((RING,)),
            pltpu.SemaphoreType.DMA,
        ],
        cost_estimate=pl.CostEstimate(
            flops=4 * E_SC * D,
            bytes_accessed=E_SC * D * 4,
            transcendentals=E_SC),
    )(edge_feats, ids_i32, wb)

    tc_out = pl.pallas_call(
        _tc_body,
        grid=(TNBLK,),
        in_specs=[
            pl.BlockSpec((TBLK, D), lambda i: (i, 0)),
            pl.BlockSpec((1, 1, TBLK), lambda i: (i, 0, 0)),
            pl.BlockSpec((D, D), lambda i: (0, 0)),
            pl.BlockSpec(memory_space=pltpu.SMEM),
        ],
        out_specs=pl.BlockSpec((G, D), lambda i: (0, 0)),
        out_shape=jax.ShapeDtypeStruct((G, D), jnp.float32),
        compiler_params=pltpu.CompilerParams(
            dimension_semantics=("arbitrary",)),
        cost_estimate=pl.CostEstimate(
            flops=2 * E_TC * D * (D + G),
            bytes_accessed=E_TC * D * 4,
            transcendentals=E_TC * D),
    )(edge_feats, ids_tc3, wrep, bvec)

    return sc_partials[0] + sc_partials[1] + tc_out


def kernel(edge_feats, edge_graph_ids, W, b):
    ids_i32 = edge_graph_ids.astype(jnp.int32)
    wb = jnp.concatenate([W[:, 0], jnp.broadcast_to(b, (L,))]).astype(jnp.float32)
    ids_tc3 = ids_i32[:E_TC].reshape(TNBLK, 1, TBLK)
    wrep = jnp.broadcast_to(W, (D, D)).astype(jnp.float32)
    bvec = b.astype(jnp.float32)
    return _cys_readout(edge_feats, ids_i32, wb, ids_tc3, wrep, bvec)
